# Initial kernel scaffold; baseline (speedup 1.0000x reference)
import jax
import jax.numpy as jnp
from jax.experimental import pallas as pl


def kernel(x, edge_index, edge_L, mask, params):
    raise NotImplementedError("write your pallas kernel here")


# speedup vs baseline: 2.9937x; 2.9937x over previous
import functools

import jax
import jax.numpy as jnp
from jax import lax
from jax.experimental import pallas as pl
from jax.experimental.pallas import tpu as pltpu
from jax.experimental.pallas import tpu_sc as plsc

NNODE = 10000
HID = 128
NCORE = 2
NSUB = 16
LANES = 16
NW = NCORE * NSUB


def _elu(v):
    return jnp.where(v > 0, v, jnp.exp(jnp.minimum(v, 0.0)) - 1.0)


def _sc_segsum(f, src, dst, lv16):
    E = src.shape[0]
    epw = E // NW
    K = 80
    nchunk = epw // K
    RPT = 640

    mesh = plsc.VectorSubcoreMesh(core_axis_name="c", subcore_axis_name="s")

    @functools.partial(
        pl.kernel,
        out_type=jax.ShapeDtypeStruct((NCORE, NNODE, HID), jnp.float32),
        mesh=mesh,
        scratch_types=[
            pltpu.VMEM((K,), jnp.int32),
            pltpu.VMEM((K,), jnp.int32),
            pltpu.VMEM((K * LANES,), jnp.float32),
            pltpu.VMEM((K, HID), jnp.float32),
            pltpu.VMEM_SHARED((NNODE, HID), jnp.float32),
            pltpu.SemaphoreType.DMA,
        ],
    )
    def k(f_hbm, src_hbm, dst_hbm, l_hbm, out_hbm,
          si_v, di_v, l_v, rows_v, acc, sem):
        cid = lax.axis_index("c")
        sid = lax.axis_index("s")
        zero16 = jnp.zeros((LANES,), jnp.float32)

        def zrow(r, carry):
            for b in range(HID // LANES):
                rows_v[r, pl.ds(b * LANES, LANES)] = zero16
            return carry

        lax.fori_loop(0, K, zrow, 0)
        r0 = pl.multiple_of(jnp.minimum(sid * (RPT - 8), NNODE - RPT), 8)
        for t in range(RPT // K):
            pltpu.sync_copy(rows_v, acc.at[pl.ds(r0 + t * K, K)])
        plsc.subcore_barrier()

        wid = cid * NSUB + sid

        def chunk(t, carry):
            off = pl.multiple_of(wid * epw + t * K, 8)
            pltpu.sync_copy(src_hbm.at[pl.ds(off, K)], si_v)
            pltpu.sync_copy(dst_hbm.at[pl.ds(off, K)], di_v)
            loff = pl.multiple_of(off * LANES, 8)
            pltpu.sync_copy(l_hbm.at[pl.ds(loff, K * LANES)], l_v)
            pltpu.async_copy(f_hbm.at[si_v], rows_v, sem).wait()

            def edge(e, c2):
                lk = l_v[pl.ds(e * LANES, LANES)]
                for b in range(HID // LANES):
                    sl = pl.ds(b * LANES, LANES)
                    rows_v[e, sl] = rows_v[e, sl] * lk
                return c2

            lax.fori_loop(0, K, edge, 0)
            pltpu.sync_copy(rows_v, acc.at[di_v], add=True)
            return carry

        lax.fori_loop(0, nchunk, chunk, 0)
        plsc.subcore_barrier()
        pltpu.sync_copy(acc.at[pl.ds(r0, RPT)], out_hbm.at[cid, pl.ds(r0, RPT)])

    return k(f, src, dst, lv16)


def _head(xp, w_t, b):
    def body(x_ref, w_ref, b_ref, o_ref):
        o_ref[...] = jnp.dot(x_ref[...], w_ref[...]) + b_ref[...]

    return pl.pallas_call(
        body, out_shape=jax.ShapeDtypeStruct((NNODE, HID), jnp.float32)
    )(xp, w_t, b)


def _lap_pre(h, w1_t, b):
    def body(h_ref, w_ref, b_ref, f_ref, p_ref):
        f = _elu(h_ref[...])
        f_ref[...] = f
        p_ref[...] = jnp.dot(f, w_ref[...]) + b_ref[...]

    return pl.pallas_call(
        body,
        out_shape=(
            jax.ShapeDtypeStruct((NNODE, HID), jnp.float32),
            jax.ShapeDtypeStruct((NNODE, HID), jnp.float32),
        ),
    )(h, w1_t, b)


def _lap_mid(p, agg, w2_t, w1n_t, bn):
    def body(p_ref, a_ref, w2_ref, w1_ref, b_ref, f_ref, pn_ref):
        a = a_ref[0] + a_ref[1]
        out = p_ref[...] + jnp.dot(a, w2_ref[...])
        f = _elu(out)
        f_ref[...] = f
        pn_ref[...] = jnp.dot(f, w1_ref[...]) + b_ref[...]

    return pl.pallas_call(
        body,
        out_shape=(
            jax.ShapeDtypeStruct((NNODE, HID), jnp.float32),
            jax.ShapeDtypeStruct((NNODE, HID), jnp.float32),
        ),
    )(p, agg, w2_t, w1n_t, bn)


def _lap_post(p, agg, w2_t, res):
    def body(p_ref, a_ref, w2_ref, r_ref, o_ref):
        a = a_ref[0] + a_ref[1]
        o_ref[...] = p_ref[...] + jnp.dot(a, w2_ref[...]) + r_ref[...]

    return pl.pallas_call(
        body, out_shape=jax.ShapeDtypeStruct((NNODE, HID), jnp.float32)
    )(p, agg, w2_t, res)


def _avg_layer(h, mask, w1t0, w2t0, b0, w1t1, w2t1, b1):
    def body(h_ref, m_ref, w10_ref, w20_ref, b0_ref, w11_ref, w21_ref,
             b1_ref, o_ref):
        m = m_ref[...]
        msum = jnp.sum(m)
        f0 = _elu(h_ref[...])
        g0 = jnp.sum(f0 * m, axis=0, keepdims=True) / msum
        out0 = (jnp.dot(f0, w10_ref[...]) + jnp.dot(g0, w20_ref[...])
                + b0_ref[...])
        f1 = _elu(out0)
        g1 = jnp.sum(f1 * m, axis=0, keepdims=True) / msum
        out1 = (jnp.dot(f1, w11_ref[...]) + jnp.dot(g1, w21_ref[...])
                + b1_ref[...])
        o_ref[...] = out1 + h_ref[...]

    return pl.pallas_call(
        body, out_shape=jax.ShapeDtypeStruct((NNODE, HID), jnp.float32)
    )(h, mask, w1t0, w2t0, b0, w1t1, w2t1, b1)


def _tail(h, wrow, b, x1):
    def body(h_ref, w_ref, b_ref, x_ref, o_ref):
        f = _elu(h_ref[...])
        o_ref[...] = (jnp.sum(f * w_ref[...], axis=1, keepdims=True)
                      + b_ref[...] + x_ref[...])

    return pl.pallas_call(
        body, out_shape=jax.ShapeDtypeStruct((NNODE, 1), jnp.float32)
    )(h, wrow, b, x1)


def kernel(x, edge_index, edge_L, mask, params):
    src = edge_index[0]
    dst = edge_index[1]
    lv16 = jnp.reshape(
        jnp.broadcast_to(edge_L, (edge_L.shape[0], LANES)), (-1,))
    x1 = x[:, :1]

    w1c, b1c = params["conv1"]
    in_f = x.shape[1]
    xp = jnp.pad(x, ((0, 0), (0, 8 - in_f)))
    w1c_t = jnp.pad(w1c.T, ((0, 8 - in_f), (0, 0)))

    h = _head(xp, w1c_t, b1c.reshape(1, HID))

    for i in range(15):
        if i % 2 == 0:
            w0, b0 = params["rn%d_fc0" % i]
            w1, b1 = params["rn%d_fc1" % i]
            res = h
            f0, p0 = _lap_pre(h, w0[:, :HID].T, b0.reshape(1, HID))
            agg0 = _sc_segsum(f0, src, dst, lv16)
            f1, p1 = _lap_mid(p0, agg0, w0[:, HID:].T, w1[:, :HID].T,
                              b1.reshape(1, HID))
            agg1 = _sc_segsum(f1, src, dst, lv16)
            h = _lap_post(p1, agg1, w1[:, HID:].T, res)
        else:
            w0, b0 = params["rn%d_fc0" % i]
            w1, b1 = params["rn%d_fc1" % i]
            h = _avg_layer(h, mask, w0[:, :HID].T, w0[:, HID:].T,
                           b0.reshape(1, HID), w1[:, :HID].T, w1[:, HID:].T,
                           b1.reshape(1, HID))

    w2c, b2c = params["conv2"]
    return _tail(h, w2c.reshape(1, HID), b2c.reshape(1, 1), x1)


# speedup vs baseline: 7.7274x; 2.5812x over previous
import functools

import jax
import jax.numpy as jnp
from jax import lax
from jax.experimental import pallas as pl
from jax.experimental.pallas import tpu as pltpu
from jax.experimental.pallas import tpu_sc as plsc

NNODE = 10000
HID = 128
NCORE = 2
NSUB = 16
LANES = 16
NW = NCORE * NSUB


def _elu(v):
    return jnp.where(v > 0, v, jnp.exp(jnp.minimum(v, 0.0)) - 1.0)


def _sc_segsum(f, src, dst, lv16):
    E = src.shape[0]
    epw = E // NW
    K = 80
    nchunk = epw // K
    RPT = 640

    last = nchunk - 1
    UNROLL = 8

    mesh = plsc.VectorSubcoreMesh(core_axis_name="c", subcore_axis_name="s")

    @functools.partial(
        pl.kernel,
        out_type=jax.ShapeDtypeStruct((NCORE, NNODE, HID), jnp.float32),
        mesh=mesh,
        scratch_types=[
            pltpu.VMEM((K,), jnp.int32), pltpu.VMEM((K,), jnp.int32),
            pltpu.VMEM((K,), jnp.int32), pltpu.VMEM((K,), jnp.int32),
            pltpu.VMEM((K * LANES,), jnp.float32),
            pltpu.VMEM((K * LANES,), jnp.float32),
            pltpu.VMEM((K, HID), jnp.float32),
            pltpu.VMEM((K, HID), jnp.float32),
            pltpu.VMEM_SHARED((NNODE, HID), jnp.float32),
            pltpu.SemaphoreType.DMA, pltpu.SemaphoreType.DMA,
            pltpu.SemaphoreType.DMA, pltpu.SemaphoreType.DMA,
            pltpu.SemaphoreType.DMA, pltpu.SemaphoreType.DMA,
            pltpu.SemaphoreType.DMA, pltpu.SemaphoreType.DMA,
        ],
    )
    def k(f_hbm, src_hbm, dst_hbm, l_hbm, out_hbm,
          si_a, si_b, di_a, di_b, l_a, l_b, rows_a, rows_b, acc,
          gsem_a, gsem_b, ssem_a, ssem_b, sisem_a, sisem_b,
          dlsem_a, dlsem_b):
        cid = lax.axis_index("c")
        sid = lax.axis_index("s")
        si = (si_a, si_b)
        di = (di_a, di_b)
        lw = (l_a, l_b)
        rows = (rows_a, rows_b)
        gsem = (gsem_a, gsem_b)
        ssem = (ssem_a, ssem_b)
        sisem = (sisem_a, sisem_b)
        dlsem = (dlsem_a, dlsem_b)
        wid = cid * NSUB + sid
        ebase = wid * epw

        def soff(ch):
            return pl.multiple_of(ebase + ch * K, 8)

        def stage_si(ch, x):
            pltpu.async_copy(src_hbm.at[pl.ds(soff(ch), K)], si[x], sisem[x])

        def wait_si(x):
            pltpu.make_async_copy(
                src_hbm.at[pl.ds(0, K)], si[x], sisem[x]).wait()

        def stage_dl(ch, x):
            pltpu.async_copy(dst_hbm.at[pl.ds(soff(ch), K)], di[x], dlsem[x])
            loff = pl.multiple_of(soff(ch) * LANES, 8)
            pltpu.async_copy(
                l_hbm.at[pl.ds(loff, K * LANES)], lw[x], dlsem[x])

        def wait_dl(x):
            pltpu.make_async_copy(
                dst_hbm.at[pl.ds(0, K)], di[x], dlsem[x]).wait()
            pltpu.make_async_copy(
                l_hbm.at[pl.ds(0, K * LANES)], lw[x], dlsem[x]).wait()

        def issue_gather(x):
            pltpu.async_copy(f_hbm.at[si[x]], rows[x], gsem[x])

        def wait_gather(x):
            pltpu.make_async_copy(f_hbm.at[si[x]], rows[x], gsem[x]).wait()

        def issue_scatter(x):
            pltpu.async_copy(rows[x], acc.at[di[x]], ssem[x], add=True)

        def wait_scatter(x):
            pltpu.make_async_copy(rows[x], acc.at[di[x]], ssem[x]).wait()

        def scale(x):
            def sbody(i, c):
                for u in range(UNROLL):
                    e = i * UNROLL + u
                    lk = lw[x][pl.ds(e * LANES, LANES)]
                    for b in range(HID // LANES):
                        sl = pl.ds(b * LANES, LANES)
                        rows[x][e, sl] = rows[x][e, sl] * lk
                return c

            lax.fori_loop(0, K // UNROLL, sbody, 0)

        zero16 = jnp.zeros((LANES,), jnp.float32)

        def zrow(r, carry):
            for b in range(HID // LANES):
                rows_a[r, pl.ds(b * LANES, LANES)] = zero16
            return carry

        lax.fori_loop(0, K, zrow, 0)
        r0 = pl.multiple_of(jnp.minimum(sid * (RPT - 8), NNODE - RPT), 8)
        for t in range(RPT // K):
            pltpu.sync_copy(rows_a, acc.at[pl.ds(r0 + t * K, K)])

        stage_si(0, 0)
        stage_dl(0, 0)
        stage_si(1, 1)
        wait_si(0)
        issue_gather(0)
        plsc.subcore_barrier()

        def phase(t, x, g):
            y = 1 - x
            wait_gather(x)
            stage_si(jnp.minimum(t + 2, last), x)

            if isinstance(g, int):
                wait_scatter(y)
            else:
                @pl.when(g >= 1)
                def _():
                    wait_scatter(y)
            stage_dl(t + 1, y)
            wait_si(y)
            issue_gather(y)
            wait_dl(x)
            scale(x)
            issue_scatter(x)

        def pair(g, carry):
            phase(2 * g, 0, g)
            phase(2 * g + 1, 1, 1)
            return carry

        lax.fori_loop(0, nchunk // 2, pair, 0)

        wait_gather(0)
        wait_scatter(1)
        wait_dl(0)
        scale(0)
        issue_scatter(0)
        wait_scatter(0)
        wait_si(1)

        plsc.subcore_barrier()
        pltpu.sync_copy(acc.at[pl.ds(r0, RPT)],
                        out_hbm.at[cid, pl.ds(r0, RPT)])

    return k(f, src, dst, lv16)


def _head(xp, w_t, b):
    def body(x_ref, w_ref, b_ref, o_ref):
        o_ref[...] = jnp.dot(x_ref[...], w_ref[...]) + b_ref[...]

    return pl.pallas_call(
        body, out_shape=jax.ShapeDtypeStruct((NNODE, HID), jnp.float32)
    )(xp, w_t, b)


def _lap_pre(h, w1_t, b):
    def body(h_ref, w_ref, b_ref, f_ref, p_ref):
        f = _elu(h_ref[...])
        f_ref[...] = f
        p_ref[...] = jnp.dot(f, w_ref[...]) + b_ref[...]

    return pl.pallas_call(
        body,
        out_shape=(
            jax.ShapeDtypeStruct((NNODE, HID), jnp.float32),
            jax.ShapeDtypeStruct((NNODE, HID), jnp.float32),
        ),
    )(h, w1_t, b)


def _lap_mid(p, agg, w2_t, w1n_t, bn):
    def body(p_ref, a_ref, w2_ref, w1_ref, b_ref, f_ref, pn_ref):
        a = a_ref[0] + a_ref[1]
        out = p_ref[...] + jnp.dot(a, w2_ref[...])
        f = _elu(out)
        f_ref[...] = f
        pn_ref[...] = jnp.dot(f, w1_ref[...]) + b_ref[...]

    return pl.pallas_call(
        body,
        out_shape=(
            jax.ShapeDtypeStruct((NNODE, HID), jnp.float32),
            jax.ShapeDtypeStruct((NNODE, HID), jnp.float32),
        ),
    )(p, agg, w2_t, w1n_t, bn)


def _lap_post(p, agg, w2_t, res):
    def body(p_ref, a_ref, w2_ref, r_ref, o_ref):
        a = a_ref[0] + a_ref[1]
        o_ref[...] = p_ref[...] + jnp.dot(a, w2_ref[...]) + r_ref[...]

    return pl.pallas_call(
        body, out_shape=jax.ShapeDtypeStruct((NNODE, HID), jnp.float32)
    )(p, agg, w2_t, res)


def _avg_layer(h, mask, w1t0, w2t0, b0, w1t1, w2t1, b1):
    def body(h_ref, m_ref, w10_ref, w20_ref, b0_ref, w11_ref, w21_ref,
             b1_ref, o_ref):
        m = m_ref[...]
        msum = jnp.sum(m)
        f0 = _elu(h_ref[...])
        g0 = jnp.sum(f0 * m, axis=0, keepdims=True) / msum
        out0 = (jnp.dot(f0, w10_ref[...]) + jnp.dot(g0, w20_ref[...])
                + b0_ref[...])
        f1 = _elu(out0)
        g1 = jnp.sum(f1 * m, axis=0, keepdims=True) / msum
        out1 = (jnp.dot(f1, w11_ref[...]) + jnp.dot(g1, w21_ref[...])
                + b1_ref[...])
        o_ref[...] = out1 + h_ref[...]

    return pl.pallas_call(
        body, out_shape=jax.ShapeDtypeStruct((NNODE, HID), jnp.float32)
    )(h, mask, w1t0, w2t0, b0, w1t1, w2t1, b1)


def _tail(h, wrow, b, x1):
    def body(h_ref, w_ref, b_ref, x_ref, o_ref):
        f = _elu(h_ref[...])
        o_ref[...] = (jnp.sum(f * w_ref[...], axis=1, keepdims=True)
                      + b_ref[...] + x_ref[...])

    return pl.pallas_call(
        body, out_shape=jax.ShapeDtypeStruct((NNODE, 1), jnp.float32)
    )(h, wrow, b, x1)


def kernel(x, edge_index, edge_L, mask, params):
    src = edge_index[0]
    dst = edge_index[1]
    lv16 = jnp.reshape(
        jnp.broadcast_to(edge_L, (edge_L.shape[0], LANES)), (-1,))
    x1 = x[:, :1]

    w1c, b1c = params["conv1"]
    in_f = x.shape[1]
    xp = jnp.pad(x, ((0, 0), (0, 8 - in_f)))
    w1c_t = jnp.pad(w1c.T, ((0, 8 - in_f), (0, 0)))

    h = _head(xp, w1c_t, b1c.reshape(1, HID))

    for i in range(15):
        if i % 2 == 0:
            w0, b0 = params["rn%d_fc0" % i]
            w1, b1 = params["rn%d_fc1" % i]
            res = h
            f0, p0 = _lap_pre(h, w0[:, :HID].T, b0.reshape(1, HID))
            agg0 = _sc_segsum(f0, src, dst, lv16)
            f1, p1 = _lap_mid(p0, agg0, w0[:, HID:].T, w1[:, :HID].T,
                              b1.reshape(1, HID))
            agg1 = _sc_segsum(f1, src, dst, lv16)
            h = _lap_post(p1, agg1, w1[:, HID:].T, res)
        else:
            w0, b0 = params["rn%d_fc0" % i]
            w1, b1 = params["rn%d_fc1" % i]
            h = _avg_layer(h, mask, w0[:, :HID].T, w0[:, HID:].T,
                           b0.reshape(1, HID), w1[:, :HID].T, w1[:, HID:].T,
                           b1.reshape(1, HID))

    w2c, b2c = params["conv2"]
    return _tail(h, w2c.reshape(1, HID), b2c.reshape(1, 1), x1)


# speedup vs baseline: 7.9357x; 1.0270x over previous
import functools

import jax
import jax.numpy as jnp
from jax import lax
from jax.experimental import pallas as pl
from jax.experimental.pallas import tpu as pltpu
from jax.experimental.pallas import tpu_sc as plsc

NNODE = 10000
HID = 128
NCORE = 2
NSUB = 16
LANES = 16
NW = NCORE * NSUB


def _elu(v):
    return jnp.where(v > 0, v, jnp.exp(jnp.minimum(v, 0.0)) - 1.0)


def _sc_segsum(f, src, dst, lv16):
    E = src.shape[0]
    epw = E // NW
    K = 128
    nchunk = epw // K
    KT = epw - nchunk * K
    RPT = 640
    UNROLL = 8

    mesh = plsc.VectorSubcoreMesh(core_axis_name="c", subcore_axis_name="s")

    @functools.partial(
        pl.kernel,
        out_type=jax.ShapeDtypeStruct((NCORE, NNODE, HID), jnp.float32),
        mesh=mesh,
        scratch_types=[
            pltpu.VMEM((K,), jnp.int32), pltpu.VMEM((K,), jnp.int32),
            pltpu.VMEM((K,), jnp.int32), pltpu.VMEM((K,), jnp.int32),
            pltpu.VMEM((K * LANES,), jnp.float32),
            pltpu.VMEM((K * LANES,), jnp.float32),
            pltpu.VMEM((K, HID), jnp.float32),
            pltpu.VMEM((K, HID), jnp.float32),
            pltpu.VMEM((KT,), jnp.int32), pltpu.VMEM((KT,), jnp.int32),
            pltpu.VMEM((KT * LANES,), jnp.float32),
            pltpu.VMEM((KT, HID), jnp.float32),
            pltpu.VMEM_SHARED((NNODE, HID), jnp.float32),
            pltpu.SemaphoreType.DMA, pltpu.SemaphoreType.DMA,
            pltpu.SemaphoreType.DMA, pltpu.SemaphoreType.DMA,
            pltpu.SemaphoreType.DMA, pltpu.SemaphoreType.DMA,
            pltpu.SemaphoreType.DMA, pltpu.SemaphoreType.DMA,
        ],
    )
    def k(f_hbm, src_hbm, dst_hbm, l_hbm, out_hbm,
          si_a, si_b, di_a, di_b, l_a, l_b, rows_a, rows_b,
          si_t, di_t, l_t, rows_t, acc,
          gsem_a, gsem_b, ssem_a, ssem_b, sisem_a, sisem_b,
          dlsem_a, dlsem_b):
        cid = lax.axis_index("c")
        sid = lax.axis_index("s")
        si = (si_a, si_b)
        di = (di_a, di_b)
        lw = (l_a, l_b)
        rows = (rows_a, rows_b)
        gsem = (gsem_a, gsem_b)
        ssem = (ssem_a, ssem_b)
        sisem = (sisem_a, sisem_b)
        dlsem = (dlsem_a, dlsem_b)
        wid = cid * NSUB + sid
        ebase = wid * epw

        def soff(ch):
            return pl.multiple_of(ebase + ch * K, 8)

        def stage_si(ch, x):
            pltpu.async_copy(src_hbm.at[pl.ds(soff(ch), K)], si[x], sisem[x])

        def wait_si(x):
            pltpu.make_async_copy(
                src_hbm.at[pl.ds(0, K)], si[x], sisem[x]).wait()

        def stage_dl(ch, x):
            pltpu.async_copy(dst_hbm.at[pl.ds(soff(ch), K)], di[x], dlsem[x])
            loff = pl.multiple_of(soff(ch) * LANES, 8)
            pltpu.async_copy(
                l_hbm.at[pl.ds(loff, K * LANES)], lw[x], dlsem[x])

        def wait_dl(x):
            pltpu.make_async_copy(
                dst_hbm.at[pl.ds(0, K)], di[x], dlsem[x]).wait()
            pltpu.make_async_copy(
                l_hbm.at[pl.ds(0, K * LANES)], lw[x], dlsem[x]).wait()

        def issue_gather(x):
            pltpu.async_copy(f_hbm.at[si[x]], rows[x], gsem[x])

        def wait_gather(x):
            pltpu.make_async_copy(f_hbm.at[si[x]], rows[x], gsem[x]).wait()

        def issue_scatter(x):
            pltpu.async_copy(rows[x], acc.at[di[x]], ssem[x], add=True)

        def wait_scatter(x):
            pltpu.make_async_copy(rows[x], acc.at[di[x]], ssem[x]).wait()

        def scale_buf(rref, lref, n):
            def sbody(i, c):
                for u in range(UNROLL):
                    e = i * UNROLL + u
                    lk = lref[pl.ds(e * LANES, LANES)]
                    for b in range(HID // LANES):
                        sl = pl.ds(b * LANES, LANES)
                        rref[e, sl] = rref[e, sl] * lk
                return c

            lax.fori_loop(0, n // UNROLL, sbody, 0)

        def scale(x):
            scale_buf(rows[x], lw[x], K)

        zero16 = jnp.zeros((LANES,), jnp.float32)

        def zrow(r, carry):
            for b in range(HID // LANES):
                rows_a[r, pl.ds(b * LANES, LANES)] = zero16
            return carry

        lax.fori_loop(0, K, zrow, 0)
        r0 = pl.multiple_of(jnp.minimum(sid * (RPT - 8), NNODE - RPT), 8)
        for t in range(RPT // K):
            pltpu.sync_copy(rows_a, acc.at[pl.ds(r0 + t * K, K)])

        stage_si(0, 0)
        stage_dl(0, 0)
        stage_si(1, 1)
        wait_si(0)
        issue_gather(0)
        plsc.subcore_barrier()

        def phase(t, x, g):
            y = 1 - x
            wait_gather(x)
            stage_si(t + 2, x)

            if isinstance(g, int):
                wait_scatter(y)
            else:
                @pl.when(g >= 1)
                def _():
                    wait_scatter(y)
            stage_dl(t + 1, y)
            wait_si(y)
            issue_gather(y)
            wait_dl(x)
            scale(x)
            issue_scatter(x)

        def pair(g, carry):
            phase(2 * g, 0, g)
            phase(2 * g + 1, 1, 1)
            return carry

        lax.fori_loop(0, (nchunk - 2) // 2, pair, 0)

        wait_gather(0)
        wait_scatter(1)
        stage_dl(nchunk - 1, 1)
        wait_si(1)
        issue_gather(1)
        wait_dl(0)
        scale(0)
        issue_scatter(0)

        wait_gather(1)
        wait_scatter(0)
        wait_dl(1)
        scale(1)
        issue_scatter(1)
        wait_scatter(1)

        tbase = pl.multiple_of(ebase + nchunk * K, 8)
        pltpu.sync_copy(src_hbm.at[pl.ds(tbase, KT)], si_t)
        pltpu.sync_copy(dst_hbm.at[pl.ds(tbase, KT)], di_t)
        tloff = pl.multiple_of(tbase * LANES, 8)
        pltpu.sync_copy(l_hbm.at[pl.ds(tloff, KT * LANES)], l_t)
        pltpu.async_copy(f_hbm.at[si_t], rows_t, gsem_a).wait()
        scale_buf(rows_t, l_t, KT)
        pltpu.sync_copy(rows_t, acc.at[di_t], add=True)

        plsc.subcore_barrier()
        pltpu.sync_copy(acc.at[pl.ds(r0, RPT)],
                        out_hbm.at[cid, pl.ds(r0, RPT)])

    return k(f, src, dst, lv16)


def _head(xp, w_t, b):
    def body(x_ref, w_ref, b_ref, o_ref):
        o_ref[...] = jnp.dot(x_ref[...], w_ref[...]) + b_ref[...]

    return pl.pallas_call(
        body, out_shape=jax.ShapeDtypeStruct((NNODE, HID), jnp.float32)
    )(xp, w_t, b)


def _lap_pre(h, w1_t, b):
    def body(h_ref, w_ref, b_ref, f_ref, p_ref):
        f = _elu(h_ref[...])
        f_ref[...] = f
        p_ref[...] = jnp.dot(f, w_ref[...]) + b_ref[...]

    return pl.pallas_call(
        body,
        out_shape=(
            jax.ShapeDtypeStruct((NNODE, HID), jnp.float32),
            jax.ShapeDtypeStruct((NNODE, HID), jnp.float32),
        ),
    )(h, w1_t, b)


def _lap_mid(p, agg, w2_t, w1n_t, bn):
    def body(p_ref, a_ref, w2_ref, w1_ref, b_ref, f_ref, pn_ref):
        a = a_ref[0] + a_ref[1]
        out = p_ref[...] + jnp.dot(a, w2_ref[...])
        f = _elu(out)
        f_ref[...] = f
        pn_ref[...] = jnp.dot(f, w1_ref[...]) + b_ref[...]

    return pl.pallas_call(
        body,
        out_shape=(
            jax.ShapeDtypeStruct((NNODE, HID), jnp.float32),
            jax.ShapeDtypeStruct((NNODE, HID), jnp.float32),
        ),
    )(p, agg, w2_t, w1n_t, bn)


def _lap_post(p, agg, w2_t, res):
    def body(p_ref, a_ref, w2_ref, r_ref, o_ref):
        a = a_ref[0] + a_ref[1]
        o_ref[...] = p_ref[...] + jnp.dot(a, w2_ref[...]) + r_ref[...]

    return pl.pallas_call(
        body, out_shape=jax.ShapeDtypeStruct((NNODE, HID), jnp.float32)
    )(p, agg, w2_t, res)


def _avg_layer(h, mask, w1t0, w2t0, b0, w1t1, w2t1, b1):
    def body(h_ref, m_ref, w10_ref, w20_ref, b0_ref, w11_ref, w21_ref,
             b1_ref, o_ref):
        m = m_ref[...]
        msum = jnp.sum(m)
        f0 = _elu(h_ref[...])
        g0 = jnp.sum(f0 * m, axis=0, keepdims=True) / msum
        out0 = (jnp.dot(f0, w10_ref[...]) + jnp.dot(g0, w20_ref[...])
                + b0_ref[...])
        f1 = _elu(out0)
        g1 = jnp.sum(f1 * m, axis=0, keepdims=True) / msum
        out1 = (jnp.dot(f1, w11_ref[...]) + jnp.dot(g1, w21_ref[...])
                + b1_ref[...])
        o_ref[...] = out1 + h_ref[...]

    return pl.pallas_call(
        body, out_shape=jax.ShapeDtypeStruct((NNODE, HID), jnp.float32)
    )(h, mask, w1t0, w2t0, b0, w1t1, w2t1, b1)


def _tail(h, wrow, b, x1):
    def body(h_ref, w_ref, b_ref, x_ref, o_ref):
        f = _elu(h_ref[...])
        o_ref[...] = (jnp.sum(f * w_ref[...], axis=1, keepdims=True)
                      + b_ref[...] + x_ref[...])

    return pl.pallas_call(
        body, out_shape=jax.ShapeDtypeStruct((NNODE, 1), jnp.float32)
    )(h, wrow, b, x1)


def kernel(x, edge_index, edge_L, mask, params):
    src = edge_index[0]
    dst = edge_index[1]
    lv16 = jnp.reshape(
        jnp.broadcast_to(edge_L, (edge_L.shape[0], LANES)), (-1,))
    x1 = x[:, :1]

    w1c, b1c = params["conv1"]
    in_f = x.shape[1]
    xp = jnp.pad(x, ((0, 0), (0, 8 - in_f)))
    w1c_t = jnp.pad(w1c.T, ((0, 8 - in_f), (0, 0)))

    h = _head(xp, w1c_t, b1c.reshape(1, HID))

    for i in range(15):
        if i % 2 == 0:
            w0, b0 = params["rn%d_fc0" % i]
            w1, b1 = params["rn%d_fc1" % i]
            res = h
            f0, p0 = _lap_pre(h, w0[:, :HID].T, b0.reshape(1, HID))
            agg0 = _sc_segsum(f0, src, dst, lv16)
            f1, p1 = _lap_mid(p0, agg0, w0[:, HID:].T, w1[:, :HID].T,
                              b1.reshape(1, HID))
            agg1 = _sc_segsum(f1, src, dst, lv16)
            h = _lap_post(p1, agg1, w1[:, HID:].T, res)
        else:
            w0, b0 = params["rn%d_fc0" % i]
            w1, b1 = params["rn%d_fc1" % i]
            h = _avg_layer(h, mask, w0[:, :HID].T, w0[:, HID:].T,
                           b0.reshape(1, HID), w1[:, :HID].T, w1[:, HID:].T,
                           b1.reshape(1, HID))

    w2c, b2c = params["conv2"]
    return _tail(h, w2c.reshape(1, HID), b2c.reshape(1, 1), x1)


# speedup vs baseline: 8.1333x; 1.0249x over previous
import functools

import jax
import jax.numpy as jnp
from jax import lax
from jax.experimental import pallas as pl
from jax.experimental.pallas import tpu as pltpu
from jax.experimental.pallas import tpu_sc as plsc

NNODE = 10000
HID = 128
NCORE = 2
NSUB = 16
LANES = 16
NW = NCORE * NSUB


def _elu(v):
    return jnp.where(v > 0, v, jnp.exp(jnp.minimum(v, 0.0)) - 1.0)


def _sc_segsum(f, src, dst, lv16):
    E = src.shape[0]
    epw = E // NW
    K = 128
    nchunk = epw // K
    KT = epw - nchunk * K
    RPT = 640
    UNROLL = 8

    mesh = plsc.VectorSubcoreMesh(core_axis_name="c", subcore_axis_name="s")

    @functools.partial(
        pl.kernel,
        out_type=jax.ShapeDtypeStruct((NCORE, NNODE, HID), jnp.float32),
        mesh=mesh,
        scratch_types=[
            pltpu.VMEM((K,), jnp.int32), pltpu.VMEM((K,), jnp.int32),
            pltpu.VMEM((K,), jnp.int32), pltpu.VMEM((K,), jnp.int32),
            pltpu.VMEM((K * LANES,), jnp.float32),
            pltpu.VMEM((K * LANES,), jnp.float32),
            pltpu.VMEM((K, HID), jnp.float32),
            pltpu.VMEM((K, HID), jnp.float32),
            pltpu.VMEM((KT,), jnp.int32), pltpu.VMEM((KT,), jnp.int32),
            pltpu.VMEM((KT * LANES,), jnp.float32),
            pltpu.VMEM((KT, HID), jnp.float32),
            pltpu.VMEM_SHARED((NNODE, HID), jnp.float32),
            pltpu.SemaphoreType.DMA, pltpu.SemaphoreType.DMA,
            pltpu.SemaphoreType.DMA, pltpu.SemaphoreType.DMA,
            pltpu.SemaphoreType.DMA, pltpu.SemaphoreType.DMA,
            pltpu.SemaphoreType.DMA, pltpu.SemaphoreType.DMA,
        ],
    )
    def k(f_hbm, src_hbm, dst_hbm, l_hbm, out_hbm,
          si_a, si_b, di_a, di_b, l_a, l_b, rows_a, rows_b,
          si_t, di_t, l_t, rows_t, acc,
          gsem_a, gsem_b, ssem_a, ssem_b, sisem_a, sisem_b,
          dlsem_a, dlsem_b):
        cid = lax.axis_index("c")
        sid = lax.axis_index("s")
        si = (si_a, si_b)
        di = (di_a, di_b)
        lw = (l_a, l_b)
        rows = (rows_a, rows_b)
        gsem = (gsem_a, gsem_b)
        ssem = (ssem_a, ssem_b)
        sisem = (sisem_a, sisem_b)
        dlsem = (dlsem_a, dlsem_b)
        wid = cid * NSUB + sid
        ebase = wid * epw

        def soff(ch):
            return pl.multiple_of(ebase + ch * K, 8)

        def stage_si(ch, x):
            pltpu.async_copy(src_hbm.at[pl.ds(soff(ch), K)], si[x], sisem[x])

        def wait_si(x):
            pltpu.make_async_copy(
                src_hbm.at[pl.ds(0, K)], si[x], sisem[x]).wait()

        def stage_dl(ch, x):
            pltpu.async_copy(dst_hbm.at[pl.ds(soff(ch), K)], di[x], dlsem[x])
            loff = pl.multiple_of(soff(ch) * LANES, 8)
            pltpu.async_copy(
                l_hbm.at[pl.ds(loff, K * LANES)], lw[x], dlsem[x])

        def wait_dl(x):
            pltpu.make_async_copy(
                dst_hbm.at[pl.ds(0, K)], di[x], dlsem[x]).wait()
            pltpu.make_async_copy(
                l_hbm.at[pl.ds(0, K * LANES)], lw[x], dlsem[x]).wait()

        def issue_gather(x):
            pltpu.async_copy(f_hbm.at[si[x]], rows[x], gsem[x])

        def wait_gather(x):
            pltpu.make_async_copy(f_hbm.at[si[x]], rows[x], gsem[x]).wait()

        def issue_scatter(x):
            pltpu.async_copy(rows[x], acc.at[di[x]], ssem[x], add=True)

        def wait_scatter(x):
            pltpu.make_async_copy(rows[x], acc.at[di[x]], ssem[x]).wait()

        def scale_buf(rref, lref, n):
            def sbody(i, c):
                for u in range(UNROLL):
                    e = i * UNROLL + u
                    lk = lref[pl.ds(e * LANES, LANES)]
                    for b in range(HID // LANES):
                        sl = pl.ds(b * LANES, LANES)
                        rref[e, sl] = rref[e, sl] * lk
                return c

            lax.fori_loop(0, n // UNROLL, sbody, 0)

        def scale(x):
            scale_buf(rows[x], lw[x], K)

        zero16 = jnp.zeros((LANES,), jnp.float32)

        def zrow(r, carry):
            for b in range(HID // LANES):
                rows_a[r, pl.ds(b * LANES, LANES)] = zero16
            return carry

        lax.fori_loop(0, K, zrow, 0)
        r0 = pl.multiple_of(jnp.minimum(sid * (RPT - 8), NNODE - RPT), 8)
        for t in range(RPT // K):
            pltpu.sync_copy(rows_a, acc.at[pl.ds(r0 + t * K, K)])

        stage_si(0, 0)
        stage_dl(0, 0)
        stage_si(1, 1)
        wait_si(0)
        issue_gather(0)
        plsc.subcore_barrier()

        def phase(t, x, g):
            y = 1 - x
            wait_gather(x)
            stage_si(t + 2, x)

            if isinstance(g, int):
                wait_scatter(y)
            else:
                @pl.when(g >= 1)
                def _():
                    wait_scatter(y)
            stage_dl(t + 1, y)
            wait_si(y)
            issue_gather(y)
            wait_dl(x)
            scale(x)
            issue_scatter(x)

        def pair(g, carry):
            phase(2 * g, 0, g)
            phase(2 * g + 1, 1, 1)
            return carry

        lax.fori_loop(0, (nchunk - 2) // 2, pair, 0)

        wait_gather(0)
        wait_scatter(1)
        stage_dl(nchunk - 1, 1)
        wait_si(1)
        issue_gather(1)
        wait_dl(0)
        scale(0)
        issue_scatter(0)

        wait_gather(1)
        wait_scatter(0)
        wait_dl(1)
        scale(1)
        issue_scatter(1)
        wait_scatter(1)

        tbase = pl.multiple_of(ebase + nchunk * K, 8)
        pltpu.sync_copy(src_hbm.at[pl.ds(tbase, KT)], si_t)
        pltpu.sync_copy(dst_hbm.at[pl.ds(tbase, KT)], di_t)
        tloff = pl.multiple_of(tbase * LANES, 8)
        pltpu.sync_copy(l_hbm.at[pl.ds(tloff, KT * LANES)], l_t)
        pltpu.async_copy(f_hbm.at[si_t], rows_t, gsem_a).wait()
        scale_buf(rows_t, l_t, KT)
        pltpu.sync_copy(rows_t, acc.at[di_t], add=True)

        plsc.subcore_barrier()
        pltpu.sync_copy(acc.at[pl.ds(r0, RPT)],
                        out_hbm.at[cid, pl.ds(r0, RPT)])

    return k(f, src, dst, lv16)


def _head_pre(xp, wc_t, bc, w1_t, b):
    def body(x_ref, wc_ref, bc_ref, w1_ref, b_ref, h_ref, f_ref, p_ref):
        h = jnp.dot(x_ref[...], wc_ref[...]) + bc_ref[...]
        h_ref[...] = h
        f = _elu(h)
        f_ref[...] = f
        p_ref[...] = jnp.dot(f, w1_ref[...]) + b_ref[...]

    s = jax.ShapeDtypeStruct((NNODE, HID), jnp.float32)
    return pl.pallas_call(body, out_shape=(s, s, s))(xp, wc_t, bc, w1_t, b)


def _lap_mid(p, agg, w2_t, w1n_t, bn):
    def body(p_ref, a_ref, w2_ref, w1_ref, b_ref, f_ref, pn_ref):
        a = a_ref[0] + a_ref[1]
        out = p_ref[...] + jnp.dot(a, w2_ref[...])
        f = _elu(out)
        f_ref[...] = f
        pn_ref[...] = jnp.dot(f, w1_ref[...]) + b_ref[...]

    s = jax.ShapeDtypeStruct((NNODE, HID), jnp.float32)
    return pl.pallas_call(body, out_shape=(s, s))(p, agg, w2_t, w1n_t, bn)


def _post_avg_pre(p, agg, w2_t, res, mask,
                  w1t0, w2t0, b0, w1t1, w2t1, b1, w1n_t, bn):
    def body(p_ref, a_ref, w2_ref, r_ref, m_ref, w10_ref, w20_ref, b0_ref,
             w11_ref, w21_ref, b1_ref, w1n_ref, bn_ref,
             h2_ref, f_ref, pn_ref):
        a = a_ref[0] + a_ref[1]
        h = p_ref[...] + jnp.dot(a, w2_ref[...]) + r_ref[...]
        m = m_ref[...]
        msum = jnp.sum(m)
        f0 = _elu(h)
        g0 = jnp.sum(f0 * m, axis=0, keepdims=True) / msum
        out0 = jnp.dot(f0, w10_ref[...]) + jnp.dot(g0, w20_ref[...]) + b0_ref[...]
        f1 = _elu(out0)
        g1 = jnp.sum(f1 * m, axis=0, keepdims=True) / msum
        out1 = jnp.dot(f1, w11_ref[...]) + jnp.dot(g1, w21_ref[...]) + b1_ref[...]
        h2 = out1 + h
        h2_ref[...] = h2
        f = _elu(h2)
        f_ref[...] = f
        pn_ref[...] = jnp.dot(f, w1n_ref[...]) + bn_ref[...]

    s = jax.ShapeDtypeStruct((NNODE, HID), jnp.float32)
    return pl.pallas_call(body, out_shape=(s, s, s))(
        p, agg, w2_t, res, mask, w1t0, w2t0, b0, w1t1, w2t1, b1, w1n_t, bn)


def _post_tail(p, agg, w2_t, res, wrow, b2, x1):
    def body(p_ref, a_ref, w2_ref, r_ref, w_ref, b_ref, x_ref, o_ref):
        a = a_ref[0] + a_ref[1]
        h = p_ref[...] + jnp.dot(a, w2_ref[...]) + r_ref[...]
        f = _elu(h)
        o_ref[...] = (jnp.sum(f * w_ref[...], axis=1, keepdims=True)
                      + b_ref[...] + x_ref[...])

    return pl.pallas_call(
        body, out_shape=jax.ShapeDtypeStruct((NNODE, 1), jnp.float32)
    )(p, agg, w2_t, res, wrow, b2, x1)


def kernel(x, edge_index, edge_L, mask, params):
    src = edge_index[0]
    dst = edge_index[1]
    lv16 = jnp.reshape(
        jnp.broadcast_to(edge_L, (edge_L.shape[0], LANES)), (-1,))
    x1 = x[:, :1]

    w1c, b1c = params["conv1"]
    in_f = x.shape[1]
    xp = jnp.pad(x, ((0, 0), (0, 8 - in_f)))
    w1c_t = jnp.pad(w1c.T, ((0, 8 - in_f), (0, 0)))

    def lapw(i):
        w0, b0 = params["rn%d_fc0" % i]
        w1, b1 = params["rn%d_fc1" % i]
        return (w0[:, :HID].T, w0[:, HID:].T, b0.reshape(1, HID),
                w1[:, :HID].T, w1[:, HID:].T, b1.reshape(1, HID))

    w10, w20, b0, w11, w21, b1 = lapw(0)
    h, f, p = _head_pre(xp, w1c_t, b1c.reshape(1, HID), w10, b0)

    for i in range(0, 15, 2):
        _, w20, _, w11, w21, b1 = lapw(i)
        agg0 = _sc_segsum(f, src, dst, lv16)
        f1, p1 = _lap_mid(p, agg0, w20, w11, b1)
        agg1 = _sc_segsum(f1, src, dst, lv16)
        if i < 14:
            aw0, ab0 = params["rn%d_fc0" % (i + 1)]
            aw1, ab1 = params["rn%d_fc1" % (i + 1)]
            nw10, _, nb0, _, _, _ = lapw(i + 2)
            h, f, p = _post_avg_pre(
                p1, agg1, w21, h, mask,
                aw0[:, :HID].T, aw0[:, HID:].T, ab0.reshape(1, HID),
                aw1[:, :HID].T, aw1[:, HID:].T, ab1.reshape(1, HID),
                nw10, nb0)
        else:
            w2c, b2c = params["conv2"]
            return _post_tail(p1, agg1, w21, h, w2c.reshape(1, HID),
                              b2c.reshape(1, 1), x1)
